# baseline (speedup 1.0000x reference)
import functools

import jax
import jax.numpy as jnp
from jax import lax
from jax.experimental import pallas as pl
from jax.experimental.pallas import tpu as pltpu
from jax.experimental.pallas import tpu_sc as plsc

_K = 8192
_D = 256
_N = 8192
_TN = 1024
_KB = 4096
_BETA = 0.25


def _argmin_body(sl_ref, lat_ref, emb_ref, se_ref, idx_ref, loss_ref):
    l = lat_ref[...]
    l2 = l + l
    sl = sl_ref[...]

    def step(c, carry):
        gmin, gidx = carry
        chunk = emb_ref[c * _KB:(c + 1) * _KB, :]
        mm2 = lax.dot_general(l2, chunk, (((1,), (1,)), ((), ())),
                             preferred_element_type=jnp.float32)
        se = se_ref[:, c * _KB:(c + 1) * _KB]
        dist = (sl + se) - mm2
        bmin = jnp.min(dist, axis=1, keepdims=True)
        ji = lax.broadcasted_iota(jnp.int32, (_TN, _KB), 1) + c * _KB
        bidx = jnp.min(jnp.where(dist == bmin, ji, jnp.int32(2**30)),
                       axis=1, keepdims=True)
        if gmin is None:
            return (bmin, bidx)
        take = bmin < gmin
        return (jnp.where(take, bmin, gmin), jnp.where(take, bidx, gidx))

    def half(c_lo, c_hi):
        carry = (None, None)
        for c in range(c_lo, c_hi):
            carry = step(c, carry)
        return carry

    nh = _K // _KB // 2
    m0, i0 = half(0, nh)
    m1, i1 = half(nh, 2 * nh)
    pick0 = m0.astype(jnp.bfloat16).astype(jnp.float32) <= m1
    idx_ref[...] = jnp.where(pick0, i0, i1)
    loss_ref[...] = jnp.where(pick0, m0, m1) * jnp.float32((1.0 + _BETA) / _D)


def _tc_argmin(sl, flat, emb, se):
    grid = (_N // _TN,)
    return pl.pallas_call(
        _argmin_body,
        grid=grid,
        in_specs=[
            pl.BlockSpec((_TN, 1), lambda i: (i, 0)),
            pl.BlockSpec((_TN, _D), lambda i: (i, 0)),
            pl.BlockSpec((_K, _D), lambda i: (0, 0)),
            pl.BlockSpec((1, _K), lambda i: (0, 0)),
        ],
        out_specs=[
            pl.BlockSpec((_TN, 1), lambda i: (i, 0)),
            pl.BlockSpec((_TN, 1), lambda i: (i, 0)),
        ],
        out_shape=[
            jax.ShapeDtypeStruct((_N, 1), jnp.int32),
            jax.ShapeDtypeStruct((_N, 1), jnp.float32),
        ],
    )(sl, flat, emb, se)


def _make_sc_gather():
    info = plsc.get_sparse_core_info()
    nw = info.num_cores * info.num_subcores
    b_per_w = _N // nw
    mesh = plsc.VectorSubcoreMesh(core_axis_name="c", subcore_axis_name="s")

    @functools.partial(
        pl.kernel,
        mesh=mesh,
        out_type=jax.ShapeDtypeStruct((_N, _D), jnp.float32),
        scratch_types=[
            pltpu.VMEM((b_per_w,), jnp.int32),
            pltpu.VMEM((b_per_w, _D), jnp.float32),
            pltpu.SemaphoreType.DMA,
        ],
    )
    def gather(table_hbm, idx_hbm, out_hbm, idx_v, rows_v, sem):
        wid = lax.axis_index("s") * info.num_cores + lax.axis_index("c")
        base = wid * b_per_w
        pltpu.sync_copy(idx_hbm.at[pl.ds(base, b_per_w)], idx_v)
        pltpu.async_copy(table_hbm.at[idx_v], rows_v, sem).wait()
        pltpu.sync_copy(rows_v, out_hbm.at[pl.ds(base, b_per_w)])

    return gather


def kernel(latents, embedding, epc):
    del epc
    b, t, d = latents.shape
    flat = latents.reshape(b * t, d)
    sl = jnp.sum(flat ** 2, axis=1, keepdims=True)
    se = jnp.sum(embedding ** 2, axis=1)[None, :]
    idx, loss = _tc_argmin(sl, flat, embedding, se)
    inds = idx.reshape(-1)
    quantized = _make_sc_gather()(embedding, inds)
    return (quantized.reshape(latents.shape), loss.reshape(b, t),
            inds.reshape(1, -1))


# speedup vs baseline: 1.0071x; 1.0071x over previous
import functools

import jax
import jax.numpy as jnp
from jax import lax
from jax.experimental import pallas as pl
from jax.experimental.pallas import tpu as pltpu
from jax.experimental.pallas import tpu_sc as plsc

_K = 8192
_D = 256
_N = 8192
_TN = 1024
_KB = 4096
_BETA = 0.25


def _argmin_body(sl_ref, lat_ref, emb_ref, se_ref, idx_ref, loss_ref):
    l = lat_ref[...]
    sl = sl_ref[...]

    def step(c, carry):
        gmin, gidx = carry
        chunk = emb_ref[c * _KB:(c + 1) * _KB, :]
        mm = lax.dot_general(l, chunk, (((1,), (1,)), ((), ())),
                             preferred_element_type=jnp.float32)
        se = se_ref[:, c * _KB:(c + 1) * _KB]
        dist = (sl + se) - 2.0 * mm
        bmin = jnp.min(dist, axis=1, keepdims=True)
        bidx = (jnp.argmin(dist, axis=1).astype(jnp.int32)
                + jnp.int32(c * _KB)).reshape(_TN, 1)
        if gmin is None:
            return (bmin, bidx)
        take = bmin < gmin
        return (jnp.where(take, bmin, gmin), jnp.where(take, bidx, gidx))

    def half(c_lo, c_hi):
        carry = (None, None)
        for c in range(c_lo, c_hi):
            carry = step(c, carry)
        return carry

    nh = _K // _KB // 2
    m0, i0 = half(0, nh)
    m1, i1 = half(nh, 2 * nh)
    pick0 = m0.astype(jnp.bfloat16).astype(jnp.float32) <= m1
    idx_ref[...] = jnp.where(pick0, i0, i1)
    loss_ref[...] = jnp.where(pick0, m0, m1) * jnp.float32((1.0 + _BETA) / _D)


def _tc_argmin(sl, flat, emb, se):
    grid = (_N // _TN,)
    return pl.pallas_call(
        _argmin_body,
        grid=grid,
        in_specs=[
            pl.BlockSpec((_TN, 1), lambda i: (i, 0)),
            pl.BlockSpec((_TN, _D), lambda i: (i, 0)),
            pl.BlockSpec((_K, _D), lambda i: (0, 0)),
            pl.BlockSpec((1, _K), lambda i: (0, 0)),
        ],
        out_specs=[
            pl.BlockSpec((_TN, 1), lambda i: (i, 0)),
            pl.BlockSpec((_TN, 1), lambda i: (i, 0)),
        ],
        out_shape=[
            jax.ShapeDtypeStruct((_N, 1), jnp.int32),
            jax.ShapeDtypeStruct((_N, 1), jnp.float32),
        ],
    )(sl, flat, emb, se)


def _make_sc_gather():
    info = plsc.get_sparse_core_info()
    nw = info.num_cores * info.num_subcores
    b_per_w = _N // nw
    mesh = plsc.VectorSubcoreMesh(core_axis_name="c", subcore_axis_name="s")

    @functools.partial(
        pl.kernel,
        mesh=mesh,
        out_type=jax.ShapeDtypeStruct((_N, _D), jnp.float32),
        scratch_types=[
            pltpu.VMEM((b_per_w,), jnp.int32),
            pltpu.VMEM((b_per_w, _D), jnp.float32),
            pltpu.SemaphoreType.DMA,
        ],
    )
    def gather(table_hbm, idx_hbm, out_hbm, idx_v, rows_v, sem):
        wid = lax.axis_index("s") * info.num_cores + lax.axis_index("c")
        base = wid * b_per_w
        pltpu.sync_copy(idx_hbm.at[pl.ds(base, b_per_w)], idx_v)
        pltpu.async_copy(table_hbm.at[idx_v], rows_v, sem).wait()
        pltpu.sync_copy(rows_v, out_hbm.at[pl.ds(base, b_per_w)])

    return gather


def kernel(latents, embedding, epc):
    del epc
    b, t, d = latents.shape
    flat = latents.reshape(b * t, d)
    sl = jnp.sum(flat ** 2, axis=1, keepdims=True)
    se = jnp.sum(embedding ** 2, axis=1)[None, :]
    idx, loss = _tc_argmin(sl, flat, embedding, se)
    inds = idx.reshape(-1)
    quantized = _make_sc_gather()(embedding, inds)
    return (quantized.reshape(latents.shape), loss.reshape(b, t),
            inds.reshape(1, -1))


# speedup vs baseline: 1.0867x; 1.0790x over previous
import functools

import jax
import jax.numpy as jnp
from jax import lax
from jax.experimental import pallas as pl
from jax.experimental.pallas import tpu as pltpu
from jax.experimental.pallas import tpu_sc as plsc

_K = 8192
_D = 256
_N = 8192
_TN = 1024
_KB = 4096
_BETA = 0.25


def _argmin_body(sl_ref, lat_ref, emb_ref, se_ref, idx_ref, loss_ref):
    l = lat_ref[...]
    sl = sl_ref[...]

    def step(c, carry):
        gmin, gidx = carry
        chunk = emb_ref[c * _KB:(c + 1) * _KB, :]
        mm = lax.dot_general(l, chunk, (((1,), (1,)), ((), ())),
                             preferred_element_type=jnp.float32)
        se = se_ref[:, c * _KB:(c + 1) * _KB]
        dist = (sl + se) - 2.0 * mm
        bmin = jnp.min(dist, axis=1, keepdims=True)
        ji = lax.broadcasted_iota(jnp.int32, (_TN, _KB), 1) + c * _KB
        bidx = jnp.min(jnp.where(dist == bmin, ji, jnp.int32(2**30)),
                       axis=1, keepdims=True)
        if gmin is None:
            return (bmin, bidx)
        take = bmin < gmin
        return (jnp.where(take, bmin, gmin), jnp.where(take, bidx, gidx))

    def half(c_lo, c_hi):
        carry = (None, None)
        for c in range(c_lo, c_hi):
            carry = step(c, carry)
        return carry

    nh = _K // _KB // 2
    m0, i0 = half(0, nh)
    m1, i1 = half(nh, 2 * nh)
    pick0 = m0.astype(jnp.bfloat16).astype(jnp.float32) <= m1
    idx_ref[...] = jnp.where(pick0, i0, i1)
    loss_ref[...] = jnp.where(pick0, m0, m1) * jnp.float32((1.0 + _BETA) / _D)


def _tc_argmin(sl, flat, emb, se):
    grid = (_N // _TN,)
    return pl.pallas_call(
        _argmin_body,
        grid=grid,
        in_specs=[
            pl.BlockSpec((_TN, 1), lambda i: (i, 0)),
            pl.BlockSpec((_TN, _D), lambda i: (i, 0)),
            pl.BlockSpec((_K, _D), lambda i: (0, 0)),
            pl.BlockSpec((1, _K), lambda i: (0, 0)),
        ],
        out_specs=[
            pl.BlockSpec((_TN, 1), lambda i: (i, 0)),
            pl.BlockSpec((_TN, 1), lambda i: (i, 0)),
        ],
        out_shape=[
            jax.ShapeDtypeStruct((_N, 1), jnp.int32),
            jax.ShapeDtypeStruct((_N, 1), jnp.float32),
        ],
    )(sl, flat, emb, se)


def _make_sc_gather():
    info = plsc.get_sparse_core_info()
    nw = info.num_cores * info.num_subcores
    b_per_w = _N // nw
    mesh = plsc.VectorSubcoreMesh(core_axis_name="c", subcore_axis_name="s")

    @functools.partial(
        pl.kernel,
        mesh=mesh,
        out_type=jax.ShapeDtypeStruct((_N, _D), jnp.float32),
        scratch_types=[
            pltpu.VMEM((b_per_w,), jnp.int32),
            pltpu.VMEM((b_per_w, _D), jnp.float32),
            pltpu.SemaphoreType.DMA,
        ],
    )
    def gather(table_hbm, idx_hbm, out_hbm, idx_v, rows_v, sem):
        wid = lax.axis_index("s") * info.num_cores + lax.axis_index("c")
        base = wid * b_per_w
        pltpu.sync_copy(idx_hbm.at[pl.ds(base, b_per_w)], idx_v)
        pltpu.async_copy(table_hbm.at[idx_v], rows_v, sem).wait()
        pltpu.sync_copy(rows_v, out_hbm.at[pl.ds(base, b_per_w)])

    return gather


def kernel(latents, embedding, epc):
    del epc
    b, t, d = latents.shape
    flat = latents.reshape(b * t, d)
    sl = jnp.sum(flat ** 2, axis=1, keepdims=True)
    se = jnp.sum(embedding ** 2, axis=1)[None, :]
    idx, loss = _tc_argmin(sl, flat, embedding, se)
    inds = idx.reshape(-1)
    quantized = _make_sc_gather()(embedding, inds)
    return (quantized.reshape(latents.shape), loss.reshape(b, t),
            inds.reshape(1, -1))


# speedup vs baseline: 1.1331x; 1.0427x over previous
import functools

import jax
import jax.numpy as jnp
from jax import lax
from jax.experimental import pallas as pl
from jax.experimental.pallas import tpu as pltpu
from jax.experimental.pallas import tpu_sc as plsc

_K = 8192
_D = 256
_N = 8192
_TN = 2048
_KB = 4096
_BETA = 0.25


def _argmin_body(sl_ref, lat_ref, emb_ref, se_ref, idx_ref, loss_ref):
    l = lat_ref[...]
    sl = sl_ref[...]

    def step(c, carry):
        gmin, gidx = carry
        chunk = emb_ref[c * _KB:(c + 1) * _KB, :]
        mm = lax.dot_general(l, chunk, (((1,), (1,)), ((), ())),
                             preferred_element_type=jnp.float32)
        se = se_ref[:, c * _KB:(c + 1) * _KB]
        dist = (sl + se) - 2.0 * mm
        bmin = jnp.min(dist, axis=1, keepdims=True)
        ji = lax.broadcasted_iota(jnp.int32, (_TN, _KB), 1) + c * _KB
        bidx = jnp.min(jnp.where(dist == bmin, ji, jnp.int32(2**30)),
                       axis=1, keepdims=True)
        if gmin is None:
            return (bmin, bidx)
        take = bmin < gmin
        return (jnp.where(take, bmin, gmin), jnp.where(take, bidx, gidx))

    def half(c_lo, c_hi):
        carry = (None, None)
        for c in range(c_lo, c_hi):
            carry = step(c, carry)
        return carry

    nh = _K // _KB // 2
    m0, i0 = half(0, nh)
    m1, i1 = half(nh, 2 * nh)
    pick0 = m0.astype(jnp.bfloat16).astype(jnp.float32) <= m1
    idx_ref[...] = jnp.where(pick0, i0, i1)
    loss_ref[...] = jnp.where(pick0, m0, m1) * jnp.float32((1.0 + _BETA) / _D)


def _tc_argmin(sl, flat, emb, se):
    grid = (_N // _TN,)
    return pl.pallas_call(
        _argmin_body,
        grid=grid,
        in_specs=[
            pl.BlockSpec((_TN, 1), lambda i: (i, 0)),
            pl.BlockSpec((_TN, _D), lambda i: (i, 0)),
            pl.BlockSpec((_K, _D), lambda i: (0, 0)),
            pl.BlockSpec((1, _K), lambda i: (0, 0)),
        ],
        out_specs=[
            pl.BlockSpec((_TN, 1), lambda i: (i, 0)),
            pl.BlockSpec((_TN, 1), lambda i: (i, 0)),
        ],
        out_shape=[
            jax.ShapeDtypeStruct((_N, 1), jnp.int32),
            jax.ShapeDtypeStruct((_N, 1), jnp.float32),
        ],
    )(sl, flat, emb, se)


def _make_sc_gather():
    info = plsc.get_sparse_core_info()
    nw = info.num_cores * info.num_subcores
    b_per_w = _N // nw
    mesh = plsc.VectorSubcoreMesh(core_axis_name="c", subcore_axis_name="s")

    @functools.partial(
        pl.kernel,
        mesh=mesh,
        out_type=jax.ShapeDtypeStruct((_N, _D), jnp.float32),
        scratch_types=[
            pltpu.VMEM((b_per_w,), jnp.int32),
            pltpu.VMEM((b_per_w, _D), jnp.float32),
            pltpu.SemaphoreType.DMA,
        ],
    )
    def gather(table_hbm, idx_hbm, out_hbm, idx_v, rows_v, sem):
        wid = lax.axis_index("s") * info.num_cores + lax.axis_index("c")
        base = wid * b_per_w
        pltpu.sync_copy(idx_hbm.at[pl.ds(base, b_per_w)], idx_v)
        pltpu.async_copy(table_hbm.at[idx_v], rows_v, sem).wait()
        pltpu.sync_copy(rows_v, out_hbm.at[pl.ds(base, b_per_w)])

    return gather


def kernel(latents, embedding, epc):
    del epc
    b, t, d = latents.shape
    flat = latents.reshape(b * t, d)
    sl = jnp.sum(flat ** 2, axis=1, keepdims=True)
    se = jnp.sum(embedding ** 2, axis=1)[None, :]
    idx, loss = _tc_argmin(sl, flat, embedding, se)
    inds = idx.reshape(-1)
    quantized = _make_sc_gather()(embedding, inds)
    return (quantized.reshape(latents.shape), loss.reshape(b, t),
            inds.reshape(1, -1))


# speedup vs baseline: 1.1718x; 1.0342x over previous
import functools

import jax
import jax.numpy as jnp
from jax import lax
from jax.experimental import pallas as pl
from jax.experimental.pallas import tpu as pltpu
from jax.experimental.pallas import tpu_sc as plsc

_K = 8192
_D = 256
_N = 8192
_TN = 2048
_KB = 4096
_BETA = 0.25


def _argmin_body(lat_ref, emb_ref, se_ref, idx_ref, loss_ref):
    l = lat_ref[...]
    sl = jnp.sum(l * l, axis=1, keepdims=True)

    def step(c, carry):
        gmin, gidx = carry
        chunk = emb_ref[c * _KB:(c + 1) * _KB, :]
        mm = lax.dot_general(l, chunk, (((1,), (1,)), ((), ())),
                             preferred_element_type=jnp.float32)
        se = se_ref[:, c * _KB:(c + 1) * _KB]
        dist = (sl + se) - 2.0 * mm
        bmin = jnp.min(dist, axis=1, keepdims=True)
        ji = lax.broadcasted_iota(jnp.int32, (_TN, _KB), 1) + c * _KB
        bidx = jnp.min(jnp.where(dist == bmin, ji, jnp.int32(2**30)),
                       axis=1, keepdims=True)
        if gmin is None:
            return (bmin, bidx)
        take = bmin < gmin
        return (jnp.where(take, bmin, gmin), jnp.where(take, bidx, gidx))

    def half(c_lo, c_hi):
        carry = (None, None)
        for c in range(c_lo, c_hi):
            carry = step(c, carry)
        return carry

    nh = _K // _KB // 2
    m0, i0 = half(0, nh)
    m1, i1 = half(nh, 2 * nh)
    pick0 = m0.astype(jnp.bfloat16).astype(jnp.float32) <= m1
    idx_ref[...] = jnp.where(pick0, i0, i1)
    loss_ref[...] = jnp.where(pick0, m0, m1) * jnp.float32((1.0 + _BETA) / _D)


def _tc_argmin(flat, emb, se):
    grid = (_N // _TN,)
    return pl.pallas_call(
        _argmin_body,
        grid=grid,
        in_specs=[
            pl.BlockSpec((_TN, _D), lambda i: (i, 0)),
            pl.BlockSpec((_K, _D), lambda i: (0, 0)),
            pl.BlockSpec((1, _K), lambda i: (0, 0)),
        ],
        out_specs=[
            pl.BlockSpec((_TN, 1), lambda i: (i, 0)),
            pl.BlockSpec((_TN, 1), lambda i: (i, 0)),
        ],
        out_shape=[
            jax.ShapeDtypeStruct((_N, 1), jnp.int32),
            jax.ShapeDtypeStruct((_N, 1), jnp.float32),
        ],
    )(flat, emb, se)


def _make_sc_gather():
    info = plsc.get_sparse_core_info()
    nw = info.num_cores * info.num_subcores
    b_per_w = _N // nw
    mesh = plsc.VectorSubcoreMesh(core_axis_name="c", subcore_axis_name="s")

    @functools.partial(
        pl.kernel,
        mesh=mesh,
        out_type=jax.ShapeDtypeStruct((_N, _D), jnp.float32),
        scratch_types=[
            pltpu.VMEM((b_per_w,), jnp.int32),
            pltpu.VMEM((b_per_w, _D), jnp.float32),
            pltpu.SemaphoreType.DMA,
        ],
    )
    def gather(table_hbm, idx_hbm, out_hbm, idx_v, rows_v, sem):
        wid = lax.axis_index("s") * info.num_cores + lax.axis_index("c")
        base = wid * b_per_w
        pltpu.sync_copy(idx_hbm.at[pl.ds(base, b_per_w)], idx_v)
        pltpu.async_copy(table_hbm.at[idx_v], rows_v, sem).wait()
        pltpu.sync_copy(rows_v, out_hbm.at[pl.ds(base, b_per_w)])

    return gather


def kernel(latents, embedding, epc):
    del epc
    b, t, d = latents.shape
    flat = latents.reshape(b * t, d)
    se = jnp.sum(embedding ** 2, axis=1)[None, :]
    idx, loss = _tc_argmin(flat, embedding, se)
    inds = idx.reshape(-1)
    quantized = _make_sc_gather()(embedding, inds)
    return (quantized.reshape(latents.shape), loss.reshape(b, t),
            inds.reshape(1, -1))
